# Initial kernel scaffold; baseline (speedup 1.0000x reference)
import jax
import jax.numpy as jnp
from jax.experimental import pallas as pl


def kernel(vid, Wg, bg, Wth, bth, Wph, bph, Ww, bw):
    raise NotImplementedError("write your pallas kernel here")


# speedup vs baseline: 13.1460x; 13.1460x over previous
import numpy as np
import jax
import jax.numpy as jnp
from jax.experimental import pallas as pl

_PS = 7
_WS = 21
_S0 = 4
_SCALE = 10.0


def _conv1x1(x, w, b):
    return jnp.einsum('tchw,oc->tohw', x, w) + b[None, :, None, None]


def _patch_vecs(f):
    T, c, H, W = f.shape
    fp = jnp.pad(f, ((0, 0), (0, 0), (0, _PS - 1), (0, _PS - 1)), mode='edge')
    cols = []
    for dy in range(_PS):
        for dx in range(_PS):
            cols.append(fp[:, :, dy:dy + H, dx:dx + W])
    P = jnp.stack(cols, axis=-1)
    P = jnp.moveaxis(P, 1, 3)
    return P.reshape(T, H, W, c * _PS * _PS)


def _score_body(pq_ref, kt_ref, cm_ref, b_ref):
    cm = cm_ref[...]
    s = jnp.dot(pq_ref[0], kt_ref[0], preferred_element_type=jnp.float32) * _SCALE
    m = jnp.max(jnp.where(cm > 0.0, s, -1e30), axis=-1, keepdims=True)
    e = cm * jnp.exp(jnp.minimum(s - m, 0.0))
    b_ref[0] = e / jnp.sum(e, axis=-1, keepdims=True)


def _wsum_body(b_ref, v_ref, zi_ref):
    zi_ref[0] = jnp.dot(b_ref[0], v_ref[0], preferred_element_type=jnp.float32)


def _make_fold_body(ic):
    def _fold_body(z_ref, ryt_ref, rx_ref, icnt_ref, yv_ref):
        ryt = ryt_ref[...]
        rx = rx_ref[...]
        icnt = icnt_ref[...]
        for ch in range(ic):
            t1 = jnp.dot(ryt, z_ref[0, ch], preferred_element_type=jnp.float32)
            yv_ref[0, ch] = jnp.dot(t1, rx, preferred_element_type=jnp.float32) * icnt
    return _fold_body


def kernel(vid, Wg, bg, Wth, bth, Wph, bph, Ww, bw):
    T, C, H, W = vid.shape
    ic = Wg.shape[0]
    nH = (H - 1) // _S0 + 1
    nW = (W - 1) // _S0 + 1
    Q = nH * nW
    D = ic * _PS * _PS
    P = H * W
    r = _WS // 2

    b1 = _conv1x1(vid, Wg, bg)
    b2 = _conv1x1(vid, Wth, bth)
    b3 = _conv1x1(vid, Wph, bph)
    Pq = _patch_vecs(b1)[:, ::_S0, ::_S0, :].reshape(T, Q, D)
    KallT = _patch_vecs(b3).reshape(T, P, D).transpose(0, 2, 1)
    Vall = _patch_vecs(b2).reshape(T, P, D)

    My = np.zeros((nH, H), np.float32)
    Mx = np.zeros((nW, W), np.float32)
    for i in range(nH):
        for dh in range(-r, r + 1):
            My[i, min(max(_S0 * i + dh, 0), H - 1)] += 1
    for j in range(nW):
        for dw in range(-r, r + 1):
            Mx[j, min(max(_S0 * j + dw, 0), W - 1)] += 1
    Cm = jnp.asarray((My[:, None, :, None] * Mx[None, :, None, :]).reshape(Q, P))

    B = pl.pallas_call(
        _score_body,
        grid=(T,),
        in_specs=[
            pl.BlockSpec((1, Q, D), lambda t: (t, 0, 0)),
            pl.BlockSpec((1, D, P), lambda t: (t, 0, 0)),
            pl.BlockSpec((Q, P), lambda t: (0, 0)),
        ],
        out_specs=pl.BlockSpec((1, Q, P), lambda t: (t, 0, 0)),
        out_shape=jax.ShapeDtypeStruct((T, Q, P), jnp.float32),
    )(Pq, KallT, Cm)

    zi = pl.pallas_call(
        _wsum_body,
        grid=(T,),
        in_specs=[
            pl.BlockSpec((1, Q, P), lambda t: (t, 0, 0)),
            pl.BlockSpec((1, P, D), lambda t: (t, 0, 0)),
        ],
        out_specs=pl.BlockSpec((1, Q, D), lambda t: (t, 0, 0)),
        out_shape=jax.ShapeDtypeStruct((T, Q, D), jnp.float32),
    )(B, Vall)

    Ry = np.zeros((nH * _PS, H), np.float32)
    Rx = np.zeros((nW * _PS, W), np.float32)
    for i in range(nH):
        for dy in range(_PS):
            Ry[i * _PS + dy, min(max(_S0 * i + dy, 0), H - 1)] += 1
    for j in range(nW):
        for dx in range(_PS):
            Rx[j * _PS + dx, min(max(_S0 * j + dx, 0), W - 1)] += 1
    icnt = jnp.asarray(1.0 / (Ry.sum(0)[:, None] * Rx.sum(0)[None, :]))
    RyT = jnp.asarray(Ry.T)
    Rxj = jnp.asarray(Rx)

    z6 = zi.reshape(T, nH, nW, ic, _PS, _PS).transpose(0, 3, 1, 4, 2, 5)
    z6 = z6.reshape(T, ic, nH * _PS, nW * _PS)

    yv = pl.pallas_call(
        _make_fold_body(ic),
        grid=(T,),
        in_specs=[
            pl.BlockSpec((1, ic, nH * _PS, nW * _PS), lambda t: (t, 0, 0, 0)),
            pl.BlockSpec((H, nH * _PS), lambda t: (0, 0)),
            pl.BlockSpec((nW * _PS, W), lambda t: (0, 0)),
            pl.BlockSpec((H, W), lambda t: (0, 0)),
        ],
        out_specs=pl.BlockSpec((1, ic, H, W), lambda t: (t, 0, 0, 0)),
        out_shape=jax.ShapeDtypeStruct((T, ic, H, W), jnp.float32),
    )(z6, RyT, Rxj, icnt)

    return vid + _conv1x1(yv, Ww, bw)
